# baseline (speedup 1.0000x reference)
import functools

import jax
import jax.numpy as jnp
from jax import lax
from jax.experimental import pallas as pl
from jax.experimental.pallas import tpu as pltpu
from jax.experimental.pallas import tpu_sc as plsc

NC = 2
NS = 16
NW = NC * NS

B_CHUNK = 128
N_LGRP = 8
N_BGRP = NW // N_LGRP


def _make_kernel(B, L, V, E, LP):
    E2 = 2 * E
    l_per_w = L // N_LGRP
    b_per_w = B // N_BGRP
    n_chunk = b_per_w // B_CHUNK
    mesh = plsc.VectorSubcoreMesh(core_axis_name="c", subcore_axis_name="s")

    @functools.partial(
        pl.kernel,
        mesh=mesh,
        out_type=jax.ShapeDtypeStruct((L, B, E2), jnp.float32),
        scratch_types=[
            pltpu.VMEM((B_CHUNK,), jnp.int32),
            pltpu.VMEM((B_CHUNK,), jnp.int32),
            pltpu.VMEM((B_CHUNK,), jnp.int32),
            pltpu.VMEM((B_CHUNK, E2), jnp.float32),
            pltpu.VMEM((B_CHUNK, E2), jnp.float32),
            pltpu.VMEM_SHARED((NS, B_CHUNK, E2), jnp.float32),
            pltpu.SemaphoreType.DMA,
        ],
        compiler_params=pltpu.CompilerParams(use_tc_tiling_on_sc=True),
    )
    def k(srcT, wt_pad, pos_pad, out,
          idx_v, ii_v, lidx_v, rows_v, posrep_v, posrep_sh, gsem):
        sid = lax.axis_index("s")
        wid = sid * NC + lax.axis_index("c")
        lg = wid // N_BGRP
        bg = wid % N_BGRP
        my_sh = posrep_sh.at[sid]
        for q in range(B_CHUNK // 16):
            ii_v[pl.ds(q * 16, 16)] = q * 16 + lax.iota(jnp.int32, 16)

        def l_body(li, c0):
            l = lg * l_per_w + li
            for q in range(B_CHUNK // 16):
                lidx_v[pl.ds(q * 16, 16)] = jnp.full((16,), l, jnp.int32)
            pltpu.async_copy(pos_pad.at[lidx_v], posrep_v, gsem).wait()

            def c_body(ci, c1):
                b0 = bg * b_per_w + ci * B_CHUNK
                pltpu.sync_copy(srcT.at[l, pl.ds(b0, B_CHUNK)], idx_v)
                gcp = pltpu.async_copy(wt_pad.at[idx_v], rows_v, gsem)
                pltpu.sync_copy(posrep_v, my_sh)
                gcp.wait()
                pltpu.sync_copy(rows_v, my_sh.at[ii_v], add=True)
                pltpu.sync_copy(my_sh, out.at[l, pl.ds(b0, B_CHUNK)])
                return c1

            lax.fori_loop(0, n_chunk, c_body, 0)
            return c0

        lax.fori_loop(0, l_per_w, l_body, 0)

    return k


def kernel(src, seg, word_table, pos_table):
    B, L = src.shape
    V, E = word_table.shape
    LP = pos_table.shape[0]
    srcT = src.T.astype(jnp.int32)
    wt_pad = jnp.pad(word_table, ((0, 0), (0, E)))
    pos_pad = jnp.pad(pos_table, ((0, 0), (0, E)))
    out = _make_kernel(B, L, V, E, LP)(srcT, wt_pad, pos_pad)
    return out[:, :, :E].transpose(1, 0, 2)


# speedup vs baseline: 1.0445x; 1.0445x over previous
import functools

import jax
import jax.numpy as jnp
from jax import lax
from jax.experimental import pallas as pl
from jax.experimental.pallas import tpu as pltpu
from jax.experimental.pallas import tpu_sc as plsc

NC = 2
NS = 16
NW = NC * NS

B_CHUNK = 128
N_LGRP = 8
N_BGRP = NW // N_LGRP


def _make_kernel(B, L, V, E, LP):
    E2 = 2 * E
    l_per_w = L // N_LGRP
    b_per_w = B // N_BGRP
    n_chunk = b_per_w // B_CHUNK
    mesh = plsc.VectorSubcoreMesh(core_axis_name="c", subcore_axis_name="s")

    @functools.partial(
        pl.kernel,
        mesh=mesh,
        out_type=jax.ShapeDtypeStruct((L, B, E2), jnp.float32),
        scratch_types=[
            pltpu.VMEM((B_CHUNK,), jnp.int32),
            pltpu.VMEM((B_CHUNK,), jnp.int32),
            pltpu.VMEM((B_CHUNK,), jnp.int32),
            pltpu.VMEM((B_CHUNK, E2), jnp.float32),
            pltpu.VMEM((B_CHUNK, E2), jnp.float32),
            pltpu.VMEM_SHARED((NS, B_CHUNK, E2), jnp.float32),
            pltpu.SemaphoreType.DMA,
            pltpu.SemaphoreType.DMA,
        ],
        compiler_params=pltpu.CompilerParams(use_tc_tiling_on_sc=True),
    )
    def k(srcT, wt_pad, pos_pad, out,
          idx_a, idx_b, lidx_v, buf_a, buf_b, posrep_sh, sem_a, sem_b):
        sid = lax.axis_index("s")
        wid = sid * NC + lax.axis_index("c")
        lg = wid // N_BGRP
        bg = wid % N_BGRP
        idx = (idx_a, idx_b)
        buf = (buf_a, buf_b)
        sem = (sem_a, sem_b)
        posrep = posrep_sh.at[sid]

        def l_body(li, c0):
            l = lg * l_per_w + li
            for q in range(B_CHUNK // 16):
                lidx_v[pl.ds(q * 16, 16)] = jnp.full((16,), l, jnp.int32)
            pltpu.async_copy(pos_pad.at[lidx_v], buf_a, sem_a).wait()
            pltpu.sync_copy(buf_a, posrep)

            def start(ci, p):
                b0 = bg * b_per_w + ci * B_CHUNK
                pltpu.sync_copy(srcT.at[l, pl.ds(b0, B_CHUNK)], idx[p])
                pltpu.sync_copy(posrep, buf[p])
                return pltpu.async_copy(wt_pad.at[idx[p]], buf[p], sem[p],
                                        add=True)

            def flush(ci, p, cp):
                b0 = bg * b_per_w + ci * B_CHUNK
                cp.wait()
                pltpu.sync_copy(buf[p], out.at[l, pl.ds(b0, B_CHUNK)])

            cp = start(0, 0)
            for ci in range(1, n_chunk):
                cp_next = start(ci, ci % 2)
                flush(ci - 1, (ci - 1) % 2, cp)
                cp = cp_next
            flush(n_chunk - 1, (n_chunk - 1) % 2, cp)
            return c0

        lax.fori_loop(0, l_per_w, l_body, 0)

    return k


def kernel(src, seg, word_table, pos_table):
    B, L = src.shape
    V, E = word_table.shape
    LP = pos_table.shape[0]
    srcT = src.T.astype(jnp.int32)
    wt_pad = jnp.pad(word_table, ((0, 0), (0, E)))
    pos_pad = jnp.pad(pos_table, ((0, 0), (0, E)))
    out = _make_kernel(B, L, V, E, LP)(srcT, wt_pad, pos_pad)
    return out[:, :, :E].transpose(1, 0, 2)


# speedup vs baseline: 1.0710x; 1.0253x over previous
import functools

import jax
import jax.numpy as jnp
from jax import lax
from jax.experimental import pallas as pl
from jax.experimental.pallas import tpu as pltpu
from jax.experimental.pallas import tpu_sc as plsc

NC = 2
NS = 16
NW = NC * NS

B_CHUNK = 256
N_LGRP = 8
N_BGRP = NW // N_LGRP


def _make_kernel(B, L, V, E, LP):
    E2 = 2 * E
    l_per_w = L // N_LGRP
    b_per_w = B // N_BGRP
    n_chunk = b_per_w // B_CHUNK
    mesh = plsc.VectorSubcoreMesh(core_axis_name="c", subcore_axis_name="s")

    @functools.partial(
        pl.kernel,
        mesh=mesh,
        out_type=jax.ShapeDtypeStruct((L, B, E2), jnp.float32),
        scratch_types=[
            pltpu.VMEM((B_CHUNK,), jnp.int32),
            pltpu.VMEM((B_CHUNK,), jnp.int32),
            pltpu.VMEM((B_CHUNK,), jnp.int32),
            pltpu.VMEM((B_CHUNK,), jnp.int32),
            pltpu.VMEM((B_CHUNK, E2), jnp.float32),
            pltpu.VMEM((B_CHUNK, E2), jnp.float32),
            pltpu.VMEM((B_CHUNK, E2), jnp.float32),
            pltpu.VMEM_SHARED((NS, 128, E2), jnp.float32),
            pltpu.SemaphoreType.DMA,
            pltpu.SemaphoreType.DMA,
            pltpu.SemaphoreType.DMA,
        ],
        compiler_params=pltpu.CompilerParams(use_tc_tiling_on_sc=True),
    )
    def k(srcT, wt_pad, pos_pad, out,
          idx_a, idx_b, idx_c, lidx_v, buf_a, buf_b, buf_c, posrep_sh,
          sem_a, sem_b, sem_c):
        sid = lax.axis_index("s")
        wid = sid * NC + lax.axis_index("c")
        lg = wid // N_BGRP
        bg = wid % N_BGRP
        idx = (idx_a, idx_b, idx_c)
        buf = (buf_a, buf_b, buf_c)
        sem = (sem_a, sem_b, sem_c)
        posrep = posrep_sh.at[sid]

        def l_body(li, c0):
            l = lg * l_per_w + li
            for q in range(B_CHUNK // 16):
                lidx_v[pl.ds(q * 16, 16)] = jnp.full((16,), l, jnp.int32)
            pltpu.async_copy(pos_pad.at[lidx_v.at[pl.ds(0, 128)]],
                             buf_a.at[pl.ds(0, 128)], sem_a).wait()
            pltpu.sync_copy(buf_a.at[pl.ds(0, 128)], posrep)

            def start(ci, p):
                b0 = bg * b_per_w + ci * B_CHUNK
                pltpu.sync_copy(srcT.at[l, pl.ds(b0, B_CHUNK)], idx[p])
                for h in range(B_CHUNK // 128):
                    pltpu.sync_copy(posrep, buf[p].at[pl.ds(h * 128, 128)])
                return pltpu.async_copy(wt_pad.at[idx[p]], buf[p], sem[p],
                                        add=True)

            def flush(ci, p, cp):
                b0 = bg * b_per_w + ci * B_CHUNK
                cp.wait()
                pltpu.sync_copy(buf[p], out.at[l, pl.ds(b0, B_CHUNK)])

            cps = [None] * n_chunk
            cps[0] = start(0, 0)
            cps[1] = start(1, 1)
            for ci in range(2, n_chunk):
                flush(ci - 2, (ci - 2) % 3, cps[ci - 2])
                cps[ci] = start(ci, ci % 3)
            flush(n_chunk - 2, (n_chunk - 2) % 3, cps[n_chunk - 2])
            flush(n_chunk - 1, (n_chunk - 1) % 3, cps[n_chunk - 1])
            return c0

        lax.fori_loop(0, l_per_w, l_body, 0)

    return k


def kernel(src, seg, word_table, pos_table):
    B, L = src.shape
    V, E = word_table.shape
    LP = pos_table.shape[0]
    srcT = src.T.astype(jnp.int32)
    wt_pad = jnp.pad(word_table, ((0, 0), (0, E)))
    pos_pad = jnp.pad(pos_table, ((0, 0), (0, E)))
    out = _make_kernel(B, L, V, E, LP)(srcT, wt_pad, pos_pad)
    return out[:, :, :E].transpose(1, 0, 2)
